# baseline (speedup 1.0000x reference)
import functools

import jax
import jax.numpy as jnp
from jax import lax
from jax.experimental import pallas as pl
from jax.experimental.pallas import tpu as pltpu
from jax.experimental.pallas import tpu_sc as plsc

NUM_HEADS = 16
NUM_TYPES = 17
B, N = 4, 512
M = N * N
RC = 8
MW = M // RC
NSUB = 16
MSUB = MW // NSUB
LANES = 16


def _sc_body(etm_hbm, wflat_hbm, out_hbm,
             wv, etm_v0, etm_v1, outb0, outb1,
             sem0, sem1):
    c = lax.axis_index("c")
    s = lax.axis_index("s")
    wid = s * 2 + c
    b = wid // RC
    rc = wid % RC

    pltpu.sync_copy(wflat_hbm, wv)

    etm_bufs = (etm_v0, etm_v1)
    out_bufs = (outb0, outb1)
    sems = (sem0, sem1)
    pending = [None, None]

    for sub in range(NSUB):
        p = sub % 2
        etm_v = etm_bufs[p]
        outb = out_bufs[p]
        if pending[p] is not None:
            pending[p].wait()
            pending[p] = None
        pltpu.sync_copy(etm_hbm.at[b, rc, pl.ds(sub * MSUB, MSUB)], etm_v)

        def body(j, carry, etm_v=etm_v, outb=outb):
            base = j * LANES
            idx = etm_v[pl.ds(base, LANES)]
            for h in range(NUM_HEADS):
                vals = plsc.load_gather(wv, [idx + h * NUM_TYPES])
                outb[h, pl.ds(base, LANES)] = vals
            return carry

        lax.fori_loop(0, MSUB // LANES, body, 0)

        off = rc * MW + sub * MSUB
        cp = pltpu.async_copy(outb, out_hbm.at[b, :, pl.ds(off, MSUB)],
                              sems[p])
        pending[p] = cp

    for p in range(2):
        if pending[p] is not None:
            pending[p].wait()


def kernel(edge_type_matrix, edge_embedding_weight):
    etm = edge_type_matrix.astype(jnp.int32).reshape(B, RC, MW)
    wflat = jnp.transpose(edge_embedding_weight, (1, 0)).reshape(-1)

    mesh = plsc.VectorSubcoreMesh(core_axis_name="c", subcore_axis_name="s")
    run = functools.partial(
        pl.kernel,
        mesh=mesh,
        compiler_params=pltpu.CompilerParams(needs_layout_passes=False),
        out_type=jax.ShapeDtypeStruct((B, NUM_HEADS, M), jnp.float32),
        scratch_types=[
            pltpu.VMEM((NUM_HEADS * NUM_TYPES,), jnp.float32),
            pltpu.VMEM((MSUB,), jnp.int32),
            pltpu.VMEM((MSUB,), jnp.int32),
            pltpu.VMEM((NUM_HEADS, MSUB), jnp.float32),
            pltpu.VMEM((NUM_HEADS, MSUB), jnp.float32),
            pltpu.SemaphoreType.DMA,
            pltpu.SemaphoreType.DMA,
        ],
    )(_sc_body)
    out = run(etm, wflat)
    return out.reshape(B, NUM_HEADS, N, N)


# speedup vs baseline: 1.9471x; 1.9471x over previous
import functools

import jax
import jax.numpy as jnp
from jax import lax
from jax.experimental import pallas as pl
from jax.experimental.pallas import tpu as pltpu
from jax.experimental.pallas import tpu_sc as plsc

NUM_HEADS = 16
NUM_TYPES = 17
TPAD = 24
B, N = 4, 512
M = N * N
RC = 8
MW = M // RC
NSUB = 16
MSUB = MW // NSUB
LANES = 16


def _sc_body(etm_hbm, wflat_hbm, out_hbm,
             wv, etm_v0, etm_v1, outb0, outb1,
             sem0, sem1):
    c = lax.axis_index("c")
    s = lax.axis_index("s")
    wid = s * 2 + c
    b = wid // RC
    rc = wid % RC

    pltpu.sync_copy(wflat_hbm, wv)

    etm_bufs = (etm_v0, etm_v1)
    out_bufs = (outb0, outb1)
    sems = (sem0, sem1)
    pending = [None, None]

    for sub in range(NSUB):
        p = sub % 2
        etm_v = etm_bufs[p]
        outb = out_bufs[p]
        if pending[p] is not None:
            pending[p].wait()
            pending[p] = None
        pltpu.sync_copy(etm_hbm.at[b, rc, pl.ds(sub * MSUB, MSUB)], etm_v)

        @plsc.parallel_loop(0, MSUB // LANES, unroll=4)
        def _(j, etm_v=etm_v, outb=outb):
            base = j * LANES
            idx = etm_v[pl.ds(base, LANES)]
            for h in range(NUM_HEADS):
                vals = plsc.load_gather(
                    wv.at[pl.ds(h * TPAD, TPAD)], [idx])
                outb[h, pl.ds(base, LANES)] = vals

        off = rc * MW + sub * MSUB
        cp = pltpu.async_copy(outb, out_hbm.at[b, :, pl.ds(off, MSUB)],
                              sems[p])
        pending[p] = cp

    for p in range(2):
        if pending[p] is not None:
            pending[p].wait()


def kernel(edge_type_matrix, edge_embedding_weight):
    etm = edge_type_matrix.astype(jnp.int32).reshape(B, RC, MW)
    wpad = jnp.zeros((NUM_HEADS, TPAD), jnp.float32)
    wpad = wpad.at[:, :NUM_TYPES].set(
        jnp.transpose(edge_embedding_weight, (1, 0)))
    wflat = wpad.reshape(-1)

    mesh = plsc.VectorSubcoreMesh(core_axis_name="c", subcore_axis_name="s")
    run = functools.partial(
        pl.kernel,
        mesh=mesh,
        compiler_params=pltpu.CompilerParams(needs_layout_passes=False),
        out_type=jax.ShapeDtypeStruct((B, NUM_HEADS, M), jnp.float32),
        scratch_types=[
            pltpu.VMEM((NUM_HEADS * TPAD,), jnp.float32),
            pltpu.VMEM((MSUB,), jnp.int32),
            pltpu.VMEM((MSUB,), jnp.int32),
            pltpu.VMEM((NUM_HEADS, MSUB), jnp.float32),
            pltpu.VMEM((NUM_HEADS, MSUB), jnp.float32),
            pltpu.SemaphoreType.DMA,
            pltpu.SemaphoreType.DMA,
        ],
    )(_sc_body)
    out = run(etm, wflat)
    return out.reshape(B, NUM_HEADS, N, N)


# speedup vs baseline: 3.9249x; 2.0158x over previous
import functools

import jax
import jax.numpy as jnp
from jax import lax
from jax.experimental import pallas as pl
from jax.experimental.pallas import tpu as pltpu
from jax.experimental.pallas import tpu_sc as plsc

NUM_HEADS = 16
NUM_TYPES = 17
TPAD = 24
B, N = 4, 512
ROWS = B * N
RC = 8
WROWS = N // RC
STRIP = 8
HHALF = NUM_HEADS // 2
LANES = 16
VPS = STRIP * N // LANES


def _sc_body(etm_hbm, wflat_hbm, out_hbm,
             wv, etm_v, outb0, outb1,
             sem0, sem1):
    c = lax.axis_index("c")
    s = lax.axis_index("s")
    wid = s * 2 + c
    b = wid // RC
    rc = wid % RC

    pltpu.sync_copy(wflat_hbm, wv)

    out_bufs = (outb0, outb1)
    sems = (sem0, sem1)
    pending = [None, None]

    for st in range(WROWS // STRIP):
        row0 = (b * N) + rc * WROWS + st * STRIP
        pltpu.sync_copy(etm_hbm.at[pl.ds(row0, STRIP), :], etm_v)

        for half in range(2):
            q = half
            outb = out_bufs[q]
            if pending[q] is not None:
                pending[q].wait()
                pending[q] = None

            @plsc.parallel_loop(0, VPS, unroll=4)
            def _(j, etm_v=etm_v, outb=outb, half=half):
                r = j // (N // LANES)
                col = (j % (N // LANES)) * LANES
                idx = etm_v[r, pl.ds(col, LANES)]
                for hh in range(HHALF):
                    h = half * HHALF + hh
                    vals = plsc.load_gather(
                        wv.at[pl.ds(h * TPAD, TPAD)], [idx])
                    outb[hh, r, pl.ds(col, LANES)] = vals

            orow = rc * WROWS + st * STRIP
            cp = pltpu.async_copy(
                outb,
                out_hbm.at[b, pl.ds(half * HHALF, HHALF),
                           pl.ds(orow, STRIP), :],
                sems[q])
            pending[q] = cp

    for p in range(2):
        if pending[p] is not None:
            pending[p].wait()


def kernel(edge_type_matrix, edge_embedding_weight):
    etm = edge_type_matrix.astype(jnp.int32).reshape(ROWS, N)
    wpad = jnp.zeros((NUM_HEADS, TPAD), jnp.float32)
    wpad = wpad.at[:, :NUM_TYPES].set(
        jnp.transpose(edge_embedding_weight, (1, 0)))
    wflat = wpad.reshape(-1)

    mesh = plsc.VectorSubcoreMesh(core_axis_name="c", subcore_axis_name="s")
    run = functools.partial(
        pl.kernel,
        mesh=mesh,
        compiler_params=pltpu.CompilerParams(needs_layout_passes=False),
        out_type=jax.ShapeDtypeStruct((B, NUM_HEADS, N, N), jnp.float32),
        scratch_types=[
            pltpu.VMEM((NUM_HEADS * TPAD,), jnp.float32),
            pltpu.VMEM((STRIP, N), jnp.int32),
            pltpu.VMEM((HHALF, STRIP, N), jnp.float32),
            pltpu.VMEM((HHALF, STRIP, N), jnp.float32),
            pltpu.SemaphoreType.DMA,
            pltpu.SemaphoreType.DMA,
        ],
    )(_sc_body)
    return run(etm, wflat)


# speedup vs baseline: 4.4915x; 1.1444x over previous
import functools

import jax
import jax.numpy as jnp
from jax import lax
from jax.experimental import pallas as pl
from jax.experimental.pallas import tpu as pltpu
from jax.experimental.pallas import tpu_sc as plsc

NUM_HEADS = 16
NUM_TYPES = 17
TPAD = 24
B, N = 4, 512
ROWS = B * N
RC = 8
WROWS = N // RC
STRIP = 8
HHALF = NUM_HEADS // 2
LANES = 16
VPS = STRIP * N // LANES


def _sc_body(etm_hbm, wflat_hbm, out_hbm,
             wv, etm_v0, etm_v1, outb0, outb1,
             sem0, sem1, esem0, esem1):
    c = lax.axis_index("c")
    s = lax.axis_index("s")
    wid = s * 2 + c
    b = wid // RC
    rc = wid % RC

    pltpu.sync_copy(wflat_hbm, wv)

    etm_bufs = (etm_v0, etm_v1)
    esems = (esem0, esem1)
    out_bufs = (outb0, outb1)
    sems = (sem0, sem1)
    pending = [None, None]
    epending = [None, None]
    nstrips = WROWS // STRIP

    def fire_etm(st):
        p = st % 2
        row0 = (b * N) + rc * WROWS + st * STRIP
        epending[p] = pltpu.async_copy(
            etm_hbm.at[pl.ds(row0, STRIP), :], etm_bufs[p], esems[p])

    fire_etm(0)
    for st in range(nstrips):
        etm_v = etm_bufs[st % 2]
        epending[st % 2].wait()
        if st + 1 < nstrips:
            fire_etm(st + 1)

        for half in range(2):
            q = half
            outb = out_bufs[q]
            if pending[q] is not None:
                pending[q].wait()
                pending[q] = None

            @plsc.parallel_loop(0, VPS, unroll=8)
            def _(j, etm_v=etm_v, outb=outb, half=half):
                r = j // (N // LANES)
                col = (j % (N // LANES)) * LANES
                idx = etm_v[r, pl.ds(col, LANES)]
                for hh in range(HHALF):
                    h = half * HHALF + hh
                    vals = plsc.load_gather(
                        wv.at[pl.ds(h * TPAD, TPAD)], [idx])
                    outb[hh, r, pl.ds(col, LANES)] = vals

            orow = rc * WROWS + st * STRIP
            cp = pltpu.async_copy(
                outb,
                out_hbm.at[b, pl.ds(half * HHALF, HHALF),
                           pl.ds(orow, STRIP), :],
                sems[q])
            pending[q] = cp

    for p in range(2):
        if pending[p] is not None:
            pending[p].wait()


def kernel(edge_type_matrix, edge_embedding_weight):
    etm = edge_type_matrix.astype(jnp.int32).reshape(ROWS, N)
    wpad = jnp.zeros((NUM_HEADS, TPAD), jnp.float32)
    wpad = wpad.at[:, :NUM_TYPES].set(
        jnp.transpose(edge_embedding_weight, (1, 0)))
    wflat = wpad.reshape(-1)

    mesh = plsc.VectorSubcoreMesh(core_axis_name="c", subcore_axis_name="s")
    run = functools.partial(
        pl.kernel,
        mesh=mesh,
        compiler_params=pltpu.CompilerParams(needs_layout_passes=False),
        out_type=jax.ShapeDtypeStruct((B, NUM_HEADS, N, N), jnp.float32),
        scratch_types=[
            pltpu.VMEM((NUM_HEADS * TPAD,), jnp.float32),
            pltpu.VMEM((STRIP, N), jnp.int32),
            pltpu.VMEM((STRIP, N), jnp.int32),
            pltpu.VMEM((HHALF, STRIP, N), jnp.float32),
            pltpu.VMEM((HHALF, STRIP, N), jnp.float32),
            pltpu.SemaphoreType.DMA,
            pltpu.SemaphoreType.DMA,
            pltpu.SemaphoreType.DMA,
            pltpu.SemaphoreType.DMA,
        ],
    )(_sc_body)
    return run(etm, wflat)
